# speedup vs baseline: 8.6623x; 8.6623x over previous
import functools

import jax
import jax.numpy as jnp
from jax.experimental import pallas as pl
from jax.experimental.pallas import tpu as pltpu

N = 16384
D = 2304
R = 53
NB = 1024

BN = 512
G = N // BN
SW = 56

_HIGH = jax.lax.Precision.HIGHEST


def _selector_kernel(x_ref, rel_ref, query_ref, starts_ref, ends_ref,
                     bias_ref, out_ref, s_scr, acc_scr, gmax_ref):
    i = pl.program_id(0)

    s_blk = jax.lax.dot_general(
        x_ref[...], rel_ref[...],
        dimension_numbers=(((1,), (0,)), ((), ())),
        preferred_element_type=jnp.float32, precision=_HIGH)
    q = query_ref[...]
    rel_ids = jax.lax.broadcasted_iota(jnp.int32, (BN, R), 1)
    att = jnp.sum(jnp.where(rel_ids == q, s_blk, 0.0), axis=1,
                  keepdims=True)
    ones = jnp.ones((BN, 1), jnp.float32)
    zeros = jnp.zeros((BN, 1), jnp.float32)
    s_scr[pl.ds(i * BN, BN), :] = jnp.concatenate(
        [s_blk, ones, att, zeros], axis=1)

    blk_max = jnp.max(att)

    @pl.when(i == 0)
    def _():
        gmax_ref[0, 0] = blk_max

    @pl.when(i > 0)
    def _():
        gmax_ref[0, 0] = jnp.maximum(gmax_ref[0, 0], blk_max)

    @pl.when(i == G - 1)
    def _():
        g = gmax_ref[0, 0]
        starts = starts_ref[...]
        ends = ends_ref[...]
        acc_scr[...] = jnp.zeros((NB, SW - 2), jnp.float32)

        def body(k, _):
            rows = pl.ds(k * BN, BN)
            tbl = s_scr[rows, :]
            att_c = tbl[:, SW - 2:SW - 1]
            e_c = jnp.exp(att_c - g)
            idx = (jax.lax.broadcasted_iota(jnp.int32, (BN, NB), 0)
                   + k * BN)
            mask = (idx >= starts) & (idx < ends)
            ew = jnp.where(mask, e_c, 0.0)
            acc_scr[...] += jax.lax.dot_general(
                ew, tbl[:, :SW - 2],
                dimension_numbers=(((0,), (0,)), ((), ())),
                preferred_element_type=jnp.float32, precision=_HIGH)
            return 0

        jax.lax.fori_loop(0, G, body, 0)
        acc = acc_scr[...]
        out_ref[...] = acc[:, :R] / acc[:, R:R + 1] + bias_ref[...]


@functools.partial(jax.jit, static_argnames=("interpret",))
def kernel(x, scope, query, rel_mat, bias, interpret=False):
    query_col = query.astype(jnp.int32).reshape(N, 1)
    starts = scope[:, 0].astype(jnp.int32).reshape(1, NB)
    ends = scope[:, 1].astype(jnp.int32).reshape(1, NB)
    bias_row = bias.reshape(1, R)

    return pl.pallas_call(
        _selector_kernel,
        grid=(G,),
        in_specs=[
            pl.BlockSpec((BN, D), lambda i: (i, 0)),
            pl.BlockSpec((D, R), lambda i: (0, 0)),
            pl.BlockSpec((BN, 1), lambda i: (i, 0)),
            pl.BlockSpec((1, NB), lambda i: (0, 0)),
            pl.BlockSpec((1, NB), lambda i: (0, 0)),
            pl.BlockSpec((1, R), lambda i: (0, 0)),
        ],
        out_specs=pl.BlockSpec((NB, R), lambda i: (0, 0)),
        out_shape=jax.ShapeDtypeStruct((NB, R), jnp.float32),
        scratch_shapes=[
            pltpu.VMEM((N, SW), jnp.float32),
            pltpu.VMEM((NB, SW - 2), jnp.float32),
            pltpu.SMEM((1, 1), jnp.float32),
        ],
        interpret=interpret,
    )(x, rel_mat, query_col, starts, ends, bias_row)


# speedup vs baseline: 19.1402x; 2.2096x over previous
import functools

import jax
import jax.numpy as jnp
from jax.experimental import pallas as pl
from jax.experimental.pallas import tpu as pltpu

N = 16384
D = 2304
R = 53
NB = 1024

BN = 512
G = N // BN
SW = 56

_HIGH = jax.lax.Precision.DEFAULT


def _selector_kernel(x_ref, rel_ref, query_ref, starts_ref, ends_ref,
                     bias_ref, out_ref, s_scr, acc_scr, gmax_ref):
    i = pl.program_id(0)

    s_blk = jax.lax.dot_general(
        x_ref[...], rel_ref[...],
        dimension_numbers=(((1,), (0,)), ((), ())),
        preferred_element_type=jnp.float32, precision=_HIGH)
    q = query_ref[...]
    rel_ids = jax.lax.broadcasted_iota(jnp.int32, (BN, R), 1)
    att = jnp.sum(jnp.where(rel_ids == q, s_blk, 0.0), axis=1,
                  keepdims=True)
    ones = jnp.ones((BN, 1), jnp.float32)
    zeros = jnp.zeros((BN, 1), jnp.float32)
    s_scr[pl.ds(i * BN, BN), :] = jnp.concatenate(
        [s_blk, ones, att, zeros], axis=1)

    blk_max = jnp.max(att)

    @pl.when(i == 0)
    def _():
        gmax_ref[0, 0] = blk_max

    @pl.when(i > 0)
    def _():
        gmax_ref[0, 0] = jnp.maximum(gmax_ref[0, 0], blk_max)

    @pl.when(i == G - 1)
    def _():
        g = gmax_ref[0, 0]
        starts = starts_ref[...]
        ends = ends_ref[...]
        acc_scr[...] = jnp.zeros((NB, SW - 2), jnp.float32)

        def body(k, _):
            rows = pl.ds(k * BN, BN)
            tbl = s_scr[rows, :]
            att_c = tbl[:, SW - 2:SW - 1]
            e_c = jnp.exp(att_c - g)
            idx = (jax.lax.broadcasted_iota(jnp.int32, (BN, NB), 0)
                   + k * BN)
            mask = (idx >= starts) & (idx < ends)
            ew = jnp.where(mask, e_c, 0.0)
            acc_scr[...] += jax.lax.dot_general(
                ew, tbl[:, :SW - 2],
                dimension_numbers=(((0,), (0,)), ((), ())),
                preferred_element_type=jnp.float32, precision=_HIGH)
            return 0

        jax.lax.fori_loop(0, G, body, 0)
        acc = acc_scr[...]
        out_ref[...] = acc[:, :R] / acc[:, R:R + 1] + bias_ref[...]


@functools.partial(jax.jit, static_argnames=("interpret",))
def kernel(x, scope, query, rel_mat, bias, interpret=False):
    query_col = query.astype(jnp.int32).reshape(N, 1)
    starts = scope[:, 0].astype(jnp.int32).reshape(1, NB)
    ends = scope[:, 1].astype(jnp.int32).reshape(1, NB)
    bias_row = bias.reshape(1, R)

    return pl.pallas_call(
        _selector_kernel,
        grid=(G,),
        in_specs=[
            pl.BlockSpec((BN, D), lambda i: (i, 0)),
            pl.BlockSpec((D, R), lambda i: (0, 0)),
            pl.BlockSpec((BN, 1), lambda i: (i, 0)),
            pl.BlockSpec((1, NB), lambda i: (0, 0)),
            pl.BlockSpec((1, NB), lambda i: (0, 0)),
            pl.BlockSpec((1, R), lambda i: (0, 0)),
        ],
        out_specs=pl.BlockSpec((NB, R), lambda i: (0, 0)),
        out_shape=jax.ShapeDtypeStruct((NB, R), jnp.float32),
        scratch_shapes=[
            pltpu.VMEM((N, SW), jnp.float32),
            pltpu.VMEM((NB, SW - 2), jnp.float32),
            pltpu.SMEM((1, 1), jnp.float32),
        ],
        interpret=interpret,
    )(x, rel_mat, query_col, starts, ends, bias_row)


# speedup vs baseline: 21.6087x; 1.1290x over previous
import functools

import jax
import jax.numpy as jnp
from jax.experimental import pallas as pl
from jax.experimental.pallas import tpu as pltpu

N = 16384
D = 2304
R = 53
NB = 1024

BN = 512
G = N // BN
TW = 54

_PREC = jax.lax.Precision.DEFAULT


def _selector_kernel(x_ref, rel_ref, query_ref, starts_ref, ends_ref,
                     bias_ref, out_ref, acc_scr, gmax_ref):
    i = pl.program_id(0)

    s_blk = jax.lax.dot_general(
        x_ref[...], rel_ref[...],
        dimension_numbers=(((1,), (0,)), ((), ())),
        preferred_element_type=jnp.float32, precision=_PREC)
    q = query_ref[...]
    rel_ids = jax.lax.broadcasted_iota(jnp.int32, (BN, R), 1)
    att = jnp.sum(jnp.where(rel_ids == q, s_blk, 0.0), axis=1,
                  keepdims=True)
    tbl = jnp.concatenate(
        [s_blk, jnp.ones((BN, 1), jnp.float32)], axis=1)

    blk_max = jnp.max(att)
    idx = jax.lax.broadcasted_iota(jnp.int32, (BN, NB), 0) + i * BN
    mask = (idx >= starts_ref[...]) & (idx < ends_ref[...])

    def contrib(m):
        ew = jnp.where(mask, jnp.exp(att - m), 0.0)
        return jax.lax.dot_general(
            ew, tbl, dimension_numbers=(((0,), (0,)), ((), ())),
            preferred_element_type=jnp.float32, precision=_PREC)

    @pl.when(i == 0)
    def _():
        gmax_ref[0, 0] = blk_max
        acc_scr[...] = contrib(blk_max)

    @pl.when(i > 0)
    def _():
        m_old = gmax_ref[0, 0]
        m_new = jnp.maximum(m_old, blk_max)
        gmax_ref[0, 0] = m_new
        acc_scr[...] = (acc_scr[...] * jnp.exp(m_old - m_new)
                        + contrib(m_new))

    @pl.when(i == G - 1)
    def _():
        acc = acc_scr[...]
        out_ref[...] = acc[:, :R] / acc[:, R:R + 1] + bias_ref[...]


@functools.partial(jax.jit, static_argnames=("interpret",))
def kernel(x, scope, query, rel_mat, bias, interpret=False):
    query_col = query.astype(jnp.int32).reshape(N, 1)
    starts = scope[:, 0].astype(jnp.int32).reshape(1, NB)
    ends = scope[:, 1].astype(jnp.int32).reshape(1, NB)
    bias_row = bias.reshape(1, R)

    return pl.pallas_call(
        _selector_kernel,
        grid=(G,),
        in_specs=[
            pl.BlockSpec((BN, D), lambda i: (i, 0)),
            pl.BlockSpec((D, R), lambda i: (0, 0)),
            pl.BlockSpec((BN, 1), lambda i: (i, 0)),
            pl.BlockSpec((1, NB), lambda i: (0, 0)),
            pl.BlockSpec((1, NB), lambda i: (0, 0)),
            pl.BlockSpec((1, R), lambda i: (0, 0)),
        ],
        out_specs=pl.BlockSpec((NB, R), lambda i: (0, 0)),
        out_shape=jax.ShapeDtypeStruct((NB, R), jnp.float32),
        scratch_shapes=[
            pltpu.VMEM((NB, TW), jnp.float32),
            pltpu.SMEM((1, 1), jnp.float32),
        ],
        interpret=interpret,
    )(x, rel_mat, query_col, starts, ends, bias_row)


# speedup vs baseline: 24.2493x; 1.1222x over previous
import functools

import jax
import jax.numpy as jnp
from jax.experimental import pallas as pl
from jax.experimental.pallas import tpu as pltpu

N = 16384
D = 2304
R = 53
NB = 1024

BN = 1024
G = N // BN
TW = 54

_PREC = jax.lax.Precision.DEFAULT


def _selector_kernel(x_ref, rel_ref, query_ref, starts_ref, ends_ref,
                     bias_ref, out_ref, acc_scr, gmax_ref):
    i = pl.program_id(0)

    s_blk = jax.lax.dot_general(
        x_ref[...], rel_ref[...],
        dimension_numbers=(((1,), (0,)), ((), ())),
        preferred_element_type=jnp.float32, precision=_PREC)
    q = query_ref[...]
    rel_ids = jax.lax.broadcasted_iota(jnp.int32, (BN, R), 1)
    att = jnp.sum(jnp.where(rel_ids == q, s_blk, 0.0), axis=1,
                  keepdims=True)
    tbl = jnp.concatenate(
        [s_blk, jnp.ones((BN, 1), jnp.float32)], axis=1)

    blk_max = jnp.max(att)
    idx = jax.lax.broadcasted_iota(jnp.int32, (BN, NB), 0) + i * BN
    mask = (idx >= starts_ref[...]) & (idx < ends_ref[...])

    def contrib(m):
        ew = jnp.where(mask, jnp.exp(att - m), 0.0)
        return jax.lax.dot_general(
            ew, tbl, dimension_numbers=(((0,), (0,)), ((), ())),
            preferred_element_type=jnp.float32, precision=_PREC)

    @pl.when(i == 0)
    def _():
        gmax_ref[0, 0] = blk_max
        acc_scr[...] = contrib(blk_max)

    @pl.when(i > 0)
    def _():
        m_old = gmax_ref[0, 0]
        m_new = jnp.maximum(m_old, blk_max)
        gmax_ref[0, 0] = m_new
        acc_scr[...] = (acc_scr[...] * jnp.exp(m_old - m_new)
                        + contrib(m_new))

    @pl.when(i == G - 1)
    def _():
        acc = acc_scr[...]
        out_ref[...] = acc[:, :R] / acc[:, R:R + 1] + bias_ref[...]


@functools.partial(jax.jit, static_argnames=("interpret",))
def kernel(x, scope, query, rel_mat, bias, interpret=False):
    query_col = query.astype(jnp.int32).reshape(N, 1)
    starts = scope[:, 0].astype(jnp.int32).reshape(1, NB)
    ends = scope[:, 1].astype(jnp.int32).reshape(1, NB)
    bias_row = bias.reshape(1, R)

    return pl.pallas_call(
        _selector_kernel,
        grid=(G,),
        in_specs=[
            pl.BlockSpec((BN, D), lambda i: (i, 0)),
            pl.BlockSpec((D, R), lambda i: (0, 0)),
            pl.BlockSpec((BN, 1), lambda i: (i, 0)),
            pl.BlockSpec((1, NB), lambda i: (0, 0)),
            pl.BlockSpec((1, NB), lambda i: (0, 0)),
            pl.BlockSpec((1, R), lambda i: (0, 0)),
        ],
        out_specs=pl.BlockSpec((NB, R), lambda i: (0, 0)),
        out_shape=jax.ShapeDtypeStruct((NB, R), jnp.float32),
        scratch_shapes=[
            pltpu.VMEM((NB, TW), jnp.float32),
            pltpu.SMEM((1, 1), jnp.float32),
        ],
        interpret=interpret,
    )(x, rel_mat, query_col, starts, ends, bias_row)


# speedup vs baseline: 24.5303x; 1.0116x over previous
import functools

import jax
import jax.numpy as jnp
from jax.experimental import pallas as pl
from jax.experimental.pallas import tpu as pltpu

N = 16384
D = 2304
R = 53
NB = 1024

BN = 2048
G = N // BN
TW = 54

_PREC = jax.lax.Precision.DEFAULT


def _selector_kernel(x_ref, rel_ref, query_ref, starts_ref, ends_ref,
                     bias_ref, out_ref, acc_scr, gmax_ref):
    i = pl.program_id(0)

    s_blk = jax.lax.dot_general(
        x_ref[...], rel_ref[...],
        dimension_numbers=(((1,), (0,)), ((), ())),
        preferred_element_type=jnp.float32, precision=_PREC)
    q = query_ref[...]
    rel_ids = jax.lax.broadcasted_iota(jnp.int32, (BN, R), 1)
    att = jnp.sum(jnp.where(rel_ids == q, s_blk, 0.0), axis=1,
                  keepdims=True)
    tbl = jnp.concatenate(
        [s_blk, jnp.ones((BN, 1), jnp.float32)], axis=1)

    blk_max = jnp.max(att)
    idx = jax.lax.broadcasted_iota(jnp.int32, (BN, NB), 0) + i * BN
    mask = (idx >= starts_ref[...]) & (idx < ends_ref[...])

    def contrib(m):
        ew = jnp.where(mask, jnp.exp(att - m), 0.0)
        return jax.lax.dot_general(
            ew, tbl, dimension_numbers=(((0,), (0,)), ((), ())),
            preferred_element_type=jnp.float32, precision=_PREC)

    @pl.when(i == 0)
    def _():
        gmax_ref[0, 0] = blk_max
        acc_scr[...] = contrib(blk_max)

    @pl.when(i > 0)
    def _():
        m_old = gmax_ref[0, 0]
        m_new = jnp.maximum(m_old, blk_max)
        gmax_ref[0, 0] = m_new
        acc_scr[...] = (acc_scr[...] * jnp.exp(m_old - m_new)
                        + contrib(m_new))

    @pl.when(i == G - 1)
    def _():
        acc = acc_scr[...]
        out_ref[...] = acc[:, :R] / acc[:, R:R + 1] + bias_ref[...]


@functools.partial(jax.jit, static_argnames=("interpret",))
def kernel(x, scope, query, rel_mat, bias, interpret=False):
    query_col = query.astype(jnp.int32).reshape(N, 1)
    starts = scope[:, 0].astype(jnp.int32).reshape(1, NB)
    ends = scope[:, 1].astype(jnp.int32).reshape(1, NB)
    bias_row = bias.reshape(1, R)

    return pl.pallas_call(
        _selector_kernel,
        grid=(G,),
        in_specs=[
            pl.BlockSpec((BN, D), lambda i: (i, 0)),
            pl.BlockSpec((D, R), lambda i: (0, 0)),
            pl.BlockSpec((BN, 1), lambda i: (i, 0)),
            pl.BlockSpec((1, NB), lambda i: (0, 0)),
            pl.BlockSpec((1, NB), lambda i: (0, 0)),
            pl.BlockSpec((1, R), lambda i: (0, 0)),
        ],
        out_specs=pl.BlockSpec((NB, R), lambda i: (0, 0)),
        out_shape=jax.ShapeDtypeStruct((NB, R), jnp.float32),
        scratch_shapes=[
            pltpu.VMEM((NB, TW), jnp.float32),
            pltpu.SMEM((1, 1), jnp.float32),
        ],
        interpret=interpret,
    )(x, rel_mat, query_col, starts, ends, bias_row)


# speedup vs baseline: 24.7809x; 1.0102x over previous
import functools

import jax
import jax.numpy as jnp
from jax.experimental import pallas as pl
from jax.experimental.pallas import tpu as pltpu

N = 16384
D = 2304
R = 53
NB = 1024

BN = 2048
G = N // BN
TW = 54

_PREC = jax.lax.Precision.DEFAULT


def _selector_kernel(x_ref, rel_ref, query_ref, starts_ref, ends_ref,
                     bias_ref, out_ref, acc_scr, gmax_ref):
    i = pl.program_id(0)

    s_blk = jax.lax.dot_general(
        x_ref[...], rel_ref[...],
        dimension_numbers=(((1,), (0,)), ((), ())),
        preferred_element_type=jnp.float32, precision=_PREC)
    q = query_ref[...]
    rel_ids = jax.lax.broadcasted_iota(jnp.int32, (BN, R), 1)
    att = jnp.sum(jnp.where(rel_ids == q, s_blk, 0.0), axis=1,
                  keepdims=True)
    tbl = jnp.concatenate(
        [s_blk, jnp.ones((BN, 1), jnp.float32)], axis=1)

    blk_max = jnp.max(att)
    idx = jax.lax.broadcasted_iota(jnp.int32, (BN, NB), 0) + i * BN
    mask = (idx >= starts_ref[...]) & (idx < ends_ref[...])

    def contrib(m):
        mbf = jnp.where(mask, jnp.float32(1.0), 0.0).astype(jnp.bfloat16)
        tbl_e = (tbl * jnp.exp(att - m)).astype(jnp.bfloat16)
        return jax.lax.dot_general(
            mbf, tbl_e, dimension_numbers=(((0,), (0,)), ((), ())),
            preferred_element_type=jnp.float32, precision=_PREC)

    @pl.when(i == 0)
    def _():
        gmax_ref[0, 0] = blk_max
        acc_scr[...] = contrib(blk_max)

    @pl.when(i > 0)
    def _():
        m_old = gmax_ref[0, 0]
        m_new = jnp.maximum(m_old, blk_max)
        gmax_ref[0, 0] = m_new
        acc_scr[...] = (acc_scr[...] * jnp.exp(m_old - m_new)
                        + contrib(m_new))

    @pl.when(i == G - 1)
    def _():
        acc = acc_scr[...]
        out_ref[...] = acc[:, :R] / acc[:, R:R + 1] + bias_ref[...]


@functools.partial(jax.jit, static_argnames=("interpret",))
def kernel(x, scope, query, rel_mat, bias, interpret=False):
    query_col = query.astype(jnp.int32).reshape(N, 1)
    starts = scope[:, 0].astype(jnp.int32).reshape(1, NB)
    ends = scope[:, 1].astype(jnp.int32).reshape(1, NB)
    bias_row = bias.reshape(1, R)

    return pl.pallas_call(
        _selector_kernel,
        grid=(G,),
        in_specs=[
            pl.BlockSpec((BN, D), lambda i: (i, 0)),
            pl.BlockSpec((D, R), lambda i: (0, 0)),
            pl.BlockSpec((BN, 1), lambda i: (i, 0)),
            pl.BlockSpec((1, NB), lambda i: (0, 0)),
            pl.BlockSpec((1, NB), lambda i: (0, 0)),
            pl.BlockSpec((1, R), lambda i: (0, 0)),
        ],
        out_specs=pl.BlockSpec((NB, R), lambda i: (0, 0)),
        out_shape=jax.ShapeDtypeStruct((NB, R), jnp.float32),
        scratch_shapes=[
            pltpu.VMEM((NB, TW), jnp.float32),
            pltpu.SMEM((1, 1), jnp.float32),
        ],
        interpret=interpret,
    )(x, rel_mat, query_col, starts, ends, bias_row)
